# baseline (speedup 1.0000x reference)
import functools

import jax
import jax.numpy as jnp
from jax import lax
from jax.experimental import pallas as pl
from jax.experimental.pallas import tpu as pltpu
from jax.experimental.pallas import tpu_sc as plsc

Q = 25000
N = 50000
E = 400000
D = 128
HALF = 64
EMB = 128

NC = 2
NS = 16
L = 16

Q_PAD = 25088
ROWS_PER_TILE_Q = Q_PAD // NS
CHUNK = 100
NCHUNK = 2
R_TILE = NCHUNK * CHUNK
E_PAD = NS * R_TILE * 128
DUMMY_HE = Q


def _sum_body(x2, pn, ph, nn, nh,
              pos_lo, pos_hi, neg_lo, neg_hi, deg_parts,
              acc, degacc, idxn, idxh, ones, zb16,
              rows_a, rows_b, rows_c, rows_d,
              sem_a, sem_b, sem_c, sem_d):
    c = lax.axis_index("c")
    s = lax.axis_index("s")
    qbase = s * ROWS_PER_TILE_Q
    slots = ((rows_a, sem_a), (rows_b, sem_b), (rows_c, sem_c),
             (rows_d, sem_d))

    zeros32 = jnp.zeros((2 * L,), jnp.bfloat16)
    zeros16f = jnp.zeros((L,), jnp.float32)
    ones16f = jnp.ones((L,), jnp.float32)

    def fill_zeros(t, _):
        rows_a[t // 2, pl.ds((t % 2) * 2 * L, 2 * L)] = zeros32
        return 0

    def fill_ones(i, _):
        ones[i, :] = ones16f
        return 0
    lax.fori_loop(0, 128, fill_ones, 0)

    def fill_zb16(i, _):
        zb16[i, :] = zeros16f
        return 0
    lax.fori_loop(0, 224, fill_zb16, 0)

    def gather(r, slot, sem):
        return pltpu.async_copy(x2.at[idxn.at[r]], slot, sem)

    def gather_wait(r, slot, sem):
        pltpu.make_async_copy(x2.at[idxn.at[r]], slot, sem).wait()

    def scat(r, slot):
        pltpu.sync_copy(slot, acc.at[idxh.at[r]], add=True)

    def count(r):
        pltpu.sync_copy(ones, degacc.at[idxh.at[r]], add=True)

    def channel_pass(nidx, hidx, out_lo, out_hi, ch):
        lax.fori_loop(0, 128 * 2, fill_zeros, 0)

        def zacc(i, _):
            pltpu.sync_copy(rows_a, acc.at[pl.ds(qbase + i * 128, 128)])
            return 0
        lax.fori_loop(0, ROWS_PER_TILE_Q // 128, zacc, 0)
        pltpu.sync_copy(
            rows_a.at[pl.ds(0, ROWS_PER_TILE_Q % 128)],
            acc.at[pl.ds(qbase + (ROWS_PER_TILE_Q // 128) * 128,
                         ROWS_PER_TILE_Q % 128)])

        def zdeg(i, _):
            pltpu.sync_copy(zb16, degacc.at[pl.ds(qbase + i * 224, 224)])
            return 0
        lax.fori_loop(0, ROWS_PER_TILE_Q // 224, zdeg, 0)

        plsc.subcore_barrier()

        def chunk_pass(k, _):
            pltpu.sync_copy(nidx.at[c].at[s].at[pl.ds(k * CHUNK, CHUNK)],
                            idxn)
            pltpu.sync_copy(hidx.at[s].at[pl.ds(k * CHUNK, CHUNK)], idxh)
            for i, (slot, sem) in enumerate(slots):
                gather(i, slot, sem)

            def quad(t, _):
                for i, (slot, sem) in enumerate(slots):
                    r = 4 * t + i
                    gather_wait(r, slot, sem)
                    scat(r, slot)
                    gather(r + 4, slot, sem)

                    @pl.when(c == k % 2)
                    def _():
                        count(r)
                return 0
            lax.fori_loop(0, CHUNK // 4 - 1, quad, 0)

            for i, (slot, sem) in enumerate(slots):
                r = CHUNK - 4 + i
                gather_wait(r, slot, sem)
                scat(r, slot)

                @pl.when(c == k % 2)
                def _():
                    count(r)
            return 0
        lax.fori_loop(0, NCHUNK, chunk_pass, 0)

        plsc.subcore_barrier()

        @pl.when(c == 0)
        def _():
            pltpu.sync_copy(acc.at[pl.ds(qbase, ROWS_PER_TILE_Q)],
                            out_lo.at[pl.ds(qbase, ROWS_PER_TILE_Q)])

        @pl.when(c == 1)
        def _():
            pltpu.sync_copy(acc.at[pl.ds(qbase, ROWS_PER_TILE_Q)],
                            out_hi.at[pl.ds(qbase, ROWS_PER_TILE_Q)])

        pltpu.sync_copy(degacc.at[pl.ds(qbase, ROWS_PER_TILE_Q)],
                        deg_parts.at[ch].at[c].at[pl.ds(qbase,
                                                        ROWS_PER_TILE_Q)])

        plsc.subcore_barrier()

    channel_pass(pn, ph, pos_lo, pos_hi, 0)
    channel_pass(nn, nh, neg_lo, neg_hi, 1)


_sum_call = functools.partial(
    pl.kernel,
    out_type=(
        jax.ShapeDtypeStruct((Q_PAD, HALF), jnp.bfloat16),
        jax.ShapeDtypeStruct((Q_PAD, HALF), jnp.bfloat16),
        jax.ShapeDtypeStruct((Q_PAD, HALF), jnp.bfloat16),
        jax.ShapeDtypeStruct((Q_PAD, HALF), jnp.bfloat16),
        jax.ShapeDtypeStruct((2, NC, Q_PAD, L), jnp.float32),
    ),
    mesh=plsc.VectorSubcoreMesh(core_axis_name="c", subcore_axis_name="s",
                                num_cores=NC, num_subcores=NS),
    compiler_params=pltpu.CompilerParams(use_tc_tiling_on_sc=False),
    scratch_types=[
        pltpu.VMEM_SHARED((Q_PAD, HALF), jnp.bfloat16),
        pltpu.VMEM_SHARED((Q_PAD, L), jnp.float32),
        pltpu.VMEM((CHUNK, 128), jnp.int32),
        pltpu.VMEM((CHUNK, 128), jnp.int32),
        pltpu.VMEM((128, L), jnp.float32),
        pltpu.VMEM((224, L), jnp.float32),
        pltpu.VMEM((128, HALF), jnp.bfloat16),
        pltpu.VMEM((128, HALF), jnp.bfloat16),
        pltpu.VMEM((128, HALF), jnp.bfloat16),
        pltpu.VMEM((128, HALF), jnp.bfloat16),
        pltpu.SemaphoreType.DMA,
        pltpu.SemaphoreType.DMA,
        pltpu.SemaphoreType.DMA,
        pltpu.SemaphoreType.DMA,
    ],
)(_sum_body)


BLK_Q = 1792


def _tc_body(xq, plo, phi, nlo, nhi, dparts, wq, bq, wn, bn, wo, bo, out):
    d = dparts[...]
    dp = d[0, :, 0] + d[1, :, 0]
    dn = d[2, :, 0] + d[3, :, 0]
    inv_p = 1.0 / jnp.maximum(dp[:, None], 1.0)
    inv_n = 1.0 / jnp.maximum(dn[:, None], 1.0)
    x = xq[...]
    wq_ = wq[...]
    wn_ = wn[...]
    wo_ = wo[...]
    plo_ = plo[...].astype(jnp.float32)
    phi_ = phi[...].astype(jnp.float32)
    nlo_ = nlo[...].astype(jnp.float32)
    nhi_ = nhi[...].astype(jnp.float32)
    hp = (jnp.dot(x, wq_[:D], preferred_element_type=jnp.float32)
          + jnp.dot(plo_ * inv_p, wq_[D:D + HALF],
                    preferred_element_type=jnp.float32)
          + jnp.dot(phi_ * inv_p, wq_[D + HALF:],
                    preferred_element_type=jnp.float32)
          + bq[...])
    hp = jnp.maximum(hp, 0.0)
    hn = (jnp.dot(x, wn_[:D], preferred_element_type=jnp.float32)
          + jnp.dot(nlo_ * inv_n, wn_[D:D + HALF],
                    preferred_element_type=jnp.float32)
          + jnp.dot(nhi_ * inv_n, wn_[D + HALF:],
                    preferred_element_type=jnp.float32)
          + bn[...])
    hn = jnp.maximum(hn, 0.0)
    out[...] = (jnp.dot(hp, wo_[:D], preferred_element_type=jnp.float32)
                + jnp.dot(hn, wo_[D:], preferred_element_type=jnp.float32)
                + bo[...])


def _tc_call(xq, plo, phi, nlo, nhi, dparts, wq, bq, wn, bn, wo, bo):
    row_spec = lambda w: pl.BlockSpec((BLK_Q, w), lambda i: (i, 0))
    full = lambda a: pl.BlockSpec(a.shape, lambda i: (0,) * a.ndim)
    return pl.pallas_call(
        _tc_body,
        grid=(Q_PAD // BLK_Q,),
        in_specs=[
            row_spec(D), row_spec(HALF), row_spec(HALF), row_spec(HALF),
            row_spec(HALF),
            pl.BlockSpec((2 * NC, BLK_Q, L), lambda i: (0, i, 0)),
            full(wq), full(bq), full(wn), full(bn), full(wo), full(bo),
        ],
        out_specs=pl.BlockSpec((BLK_Q, EMB), lambda i: (i, 0)),
        out_shape=jax.ShapeDtypeStruct((Q_PAD, EMB), jnp.float32),
    )(xq, plo, phi, nlo, nhi, dparts, wq, bq, wn, bn, wo, bo)


def _prep_he(idx):
    idx = idx.astype(jnp.int32)
    pad = jnp.full((E_PAD - E,), DUMMY_HE, jnp.int32)
    return jnp.concatenate([idx, pad]).reshape(NS, R_TILE, 128)


def _prep_node(idx):
    idx = idx.astype(jnp.int32)
    pad = jnp.zeros((E_PAD - E,), jnp.int32)
    idx2 = 2 * jnp.concatenate([idx, pad])
    return jnp.stack([idx2, idx2 + 1]).reshape(NC, NS, R_TILE, 128)


def kernel(x, pos_node_idx, pos_he_idx, neg_node_idx, neg_he_idx,
           W_q, b_q, W_n, b_n, W_out, b_out):
    x2 = x.astype(jnp.bfloat16).reshape(2 * N, HALF)
    pn = _prep_node(pos_node_idx)
    ph = _prep_he(pos_he_idx)
    nn = _prep_node(neg_node_idx)
    nh = _prep_he(neg_he_idx)

    pos_lo, pos_hi, neg_lo, neg_hi, deg_parts = _sum_call(x2, pn, ph, nn, nh)

    out = _tc_call(x, pos_lo, pos_hi, neg_lo, neg_hi,
                   deg_parts.reshape(2 * NC, Q_PAD, L),
                   W_q, b_q.reshape(1, D), W_n, b_n.reshape(1, D),
                   W_out, b_out.reshape(1, EMB))
    return out[:Q]


# speedup vs baseline: 1.0313x; 1.0313x over previous
import functools

import jax
import jax.numpy as jnp
from jax import lax
from jax.experimental import pallas as pl
from jax.experimental.pallas import tpu as pltpu
from jax.experimental.pallas import tpu_sc as plsc

Q = 25000
N = 50000
E = 400000
D = 128
HALF = 64
EMB = 128

NC = 2
NS = 16
L = 16

Q_PAD = 25088
ROWS_PER_TILE_Q = Q_PAD // NS
CHUNK = 100
NCHUNK = 2
R_TILE = NCHUNK * CHUNK
E_PAD = NS * R_TILE * 128
DUMMY_HE = Q


def _sum_body(x2, pn, ph, nn, nh,
              pos_lo, pos_hi, neg_lo, neg_hi,
              acc, idxn, idxh, rows_a, rows_b, rows_c, rows_d,
              sem_a, sem_b, sem_c, sem_d):
    c = lax.axis_index("c")
    s = lax.axis_index("s")
    qbase = s * ROWS_PER_TILE_Q
    slots = ((rows_a, sem_a), (rows_b, sem_b), (rows_c, sem_c),
             (rows_d, sem_d))

    zeros32 = jnp.zeros((2 * L,), jnp.bfloat16)
    zeros16f = jnp.zeros((L,), jnp.float32)
    ones16f = jnp.ones((L,), jnp.float32)

    def fill_zeros(t, _):
        rows_a[t // 2, pl.ds((t % 2) * 2 * L, 2 * L)] = zeros32
        return 0


    def gather(r, slot, sem):
        return pltpu.async_copy(x2.at[idxn.at[r]], slot, sem)

    def gather_wait(r, slot, sem):
        pltpu.make_async_copy(x2.at[idxn.at[r]], slot, sem).wait()

    def scat(r, slot):
        pltpu.sync_copy(slot, acc.at[idxh.at[r]], add=True)


    def channel_pass(nidx, hidx, out_lo, out_hi):
        lax.fori_loop(0, 128 * 2, fill_zeros, 0)

        def zacc(i, _):
            pltpu.sync_copy(rows_a, acc.at[pl.ds(qbase + i * 128, 128)])
            return 0
        lax.fori_loop(0, ROWS_PER_TILE_Q // 128, zacc, 0)
        pltpu.sync_copy(
            rows_a.at[pl.ds(0, ROWS_PER_TILE_Q % 128)],
            acc.at[pl.ds(qbase + (ROWS_PER_TILE_Q // 128) * 128,
                         ROWS_PER_TILE_Q % 128)])

        plsc.subcore_barrier()

        pltpu.sync_copy(nidx.at[c].at[s], idxn)
        pltpu.sync_copy(hidx.at[s], idxh)
        for i, (slot, sem) in enumerate(slots):
            gather(i, slot, sem)

        def quad(t, _):
            for i, (slot, sem) in enumerate(slots):
                r = 4 * t + i
                gather_wait(r, slot, sem)
                scat(r, slot)
                gather(r + 4, slot, sem)
            return 0
        lax.fori_loop(0, R_TILE // 4 - 1, quad, 0)

        for i, (slot, sem) in enumerate(slots):
            r = R_TILE - 4 + i
            gather_wait(r, slot, sem)
            scat(r, slot)

        plsc.subcore_barrier()

        @pl.when(c == 0)
        def _():
            pltpu.sync_copy(acc.at[pl.ds(qbase, ROWS_PER_TILE_Q)],
                            out_lo.at[pl.ds(qbase, ROWS_PER_TILE_Q)])

        @pl.when(c == 1)
        def _():
            pltpu.sync_copy(acc.at[pl.ds(qbase, ROWS_PER_TILE_Q)],
                            out_hi.at[pl.ds(qbase, ROWS_PER_TILE_Q)])

        plsc.subcore_barrier()

    channel_pass(pn, ph, pos_lo, pos_hi)
    channel_pass(nn, nh, neg_lo, neg_hi)


_sum_call = functools.partial(
    pl.kernel,
    out_type=(
        jax.ShapeDtypeStruct((Q_PAD, HALF), jnp.bfloat16),
        jax.ShapeDtypeStruct((Q_PAD, HALF), jnp.bfloat16),
        jax.ShapeDtypeStruct((Q_PAD, HALF), jnp.bfloat16),
        jax.ShapeDtypeStruct((Q_PAD, HALF), jnp.bfloat16),
    ),
    mesh=plsc.VectorSubcoreMesh(core_axis_name="c", subcore_axis_name="s",
                                num_cores=NC, num_subcores=NS),
    compiler_params=pltpu.CompilerParams(use_tc_tiling_on_sc=False),
    scratch_types=[
        pltpu.VMEM_SHARED((Q_PAD, HALF), jnp.bfloat16),
        pltpu.VMEM((R_TILE, 128), jnp.int32),
        pltpu.VMEM((R_TILE, 128), jnp.int32),
        pltpu.VMEM((128, HALF), jnp.bfloat16),
        pltpu.VMEM((128, HALF), jnp.bfloat16),
        pltpu.VMEM((128, HALF), jnp.bfloat16),
        pltpu.VMEM((128, HALF), jnp.bfloat16),
        pltpu.SemaphoreType.DMA,
        pltpu.SemaphoreType.DMA,
        pltpu.SemaphoreType.DMA,
        pltpu.SemaphoreType.DMA,
    ],
)(_sum_body)


def _deg_body(ph, nh, deg_pos, deg_neg, degacc, idxh, ones, zb16):
    c = lax.axis_index("c")
    s = lax.axis_index("s")
    qbase = s * ROWS_PER_TILE_Q

    zeros16 = jnp.zeros((L,), jnp.float32)
    ones16 = jnp.ones((L,), jnp.float32)

    def fill_ones(i, _):
        ones[i, :] = ones16
        return 0
    lax.fori_loop(0, 128, fill_ones, 0)

    def fill_zb16(i, _):
        zb16[i, :] = zeros16
        return 0
    lax.fori_loop(0, 224, fill_zb16, 0)

    def zdeg(i, _):
        pltpu.sync_copy(zb16, degacc.at[pl.ds(qbase + i * 224, 224)])
        return 0
    lax.fori_loop(0, ROWS_PER_TILE_Q // 224, zdeg, 0)

    plsc.subcore_barrier()

    @pl.when(c == 0)
    def _():
        pltpu.sync_copy(ph.at[s], idxh)

    @pl.when(c == 1)
    def _():
        pltpu.sync_copy(nh.at[s], idxh)

    def step(j, _):
        pltpu.sync_copy(ones, degacc.at[idxh.at[j]], add=True)
        return 0
    lax.fori_loop(0, R_TILE, step, 0)

    plsc.subcore_barrier()

    @pl.when(c == 0)
    def _():
        pltpu.sync_copy(degacc.at[pl.ds(qbase, ROWS_PER_TILE_Q)],
                        deg_pos.at[pl.ds(qbase, ROWS_PER_TILE_Q)])

    @pl.when(c == 1)
    def _():
        pltpu.sync_copy(degacc.at[pl.ds(qbase, ROWS_PER_TILE_Q)],
                        deg_neg.at[pl.ds(qbase, ROWS_PER_TILE_Q)])


_deg_call = functools.partial(
    pl.kernel,
    out_type=(
        jax.ShapeDtypeStruct((Q_PAD, L), jnp.float32),
        jax.ShapeDtypeStruct((Q_PAD, L), jnp.float32),
    ),
    mesh=plsc.VectorSubcoreMesh(core_axis_name="c", subcore_axis_name="s",
                                num_cores=NC, num_subcores=NS),
    compiler_params=pltpu.CompilerParams(use_tc_tiling_on_sc=False),
    scratch_types=[
        pltpu.VMEM_SHARED((Q_PAD, L), jnp.float32),
        pltpu.VMEM((R_TILE, 128), jnp.int32),
        pltpu.VMEM((128, L), jnp.float32),
        pltpu.VMEM((224, L), jnp.float32),
    ],
)(_deg_body)


BLK_Q = 1792


def _tc_body(xq, plo, phi, nlo, nhi, dp, dn, wq, bq, wn, bn, wo, bo, out):
    inv_p = 1.0 / jnp.maximum(dp[:, 0:1], 1.0)
    inv_n = 1.0 / jnp.maximum(dn[:, 0:1], 1.0)
    x = xq[...]
    wq_ = wq[...]
    wn_ = wn[...]
    wo_ = wo[...]
    plo_ = plo[...].astype(jnp.float32)
    phi_ = phi[...].astype(jnp.float32)
    nlo_ = nlo[...].astype(jnp.float32)
    nhi_ = nhi[...].astype(jnp.float32)
    hp = (jnp.dot(x, wq_[:D], preferred_element_type=jnp.float32)
          + jnp.dot(plo_ * inv_p, wq_[D:D + HALF],
                    preferred_element_type=jnp.float32)
          + jnp.dot(phi_ * inv_p, wq_[D + HALF:],
                    preferred_element_type=jnp.float32)
          + bq[...])
    hp = jnp.maximum(hp, 0.0)
    hn = (jnp.dot(x, wn_[:D], preferred_element_type=jnp.float32)
          + jnp.dot(nlo_ * inv_n, wn_[D:D + HALF],
                    preferred_element_type=jnp.float32)
          + jnp.dot(nhi_ * inv_n, wn_[D + HALF:],
                    preferred_element_type=jnp.float32)
          + bn[...])
    hn = jnp.maximum(hn, 0.0)
    out[...] = (jnp.dot(hp, wo_[:D], preferred_element_type=jnp.float32)
                + jnp.dot(hn, wo_[D:], preferred_element_type=jnp.float32)
                + bo[...])


def _tc_call(xq, plo, phi, nlo, nhi, dp, dn, wq, bq, wn, bn, wo, bo):
    row_spec = lambda w: pl.BlockSpec((BLK_Q, w), lambda i: (i, 0))
    full = lambda a: pl.BlockSpec(a.shape, lambda i: (0,) * a.ndim)
    return pl.pallas_call(
        _tc_body,
        grid=(Q_PAD // BLK_Q,),
        in_specs=[
            row_spec(D), row_spec(HALF), row_spec(HALF), row_spec(HALF),
            row_spec(HALF), row_spec(L), row_spec(L),
            full(wq), full(bq), full(wn), full(bn), full(wo), full(bo),
        ],
        out_specs=pl.BlockSpec((BLK_Q, EMB), lambda i: (i, 0)),
        out_shape=jax.ShapeDtypeStruct((Q, EMB), jnp.float32),
    )(xq, plo, phi, nlo, nhi, dp, dn, wq, bq, wn, bn, wo, bo)


def _prep_he(idx):
    idx = idx.astype(jnp.int32)
    pad = jnp.full((E_PAD - E,), DUMMY_HE, jnp.int32)
    return jnp.concatenate([idx, pad]).reshape(NS, R_TILE, 128)


def _prep_node(idx):
    idx = idx.astype(jnp.int32)
    pad = jnp.zeros((E_PAD - E,), jnp.int32)
    idx2 = 2 * jnp.concatenate([idx, pad])
    return jnp.stack([idx2, idx2 + 1]).reshape(NC, NS, R_TILE, 128)


def kernel(x, pos_node_idx, pos_he_idx, neg_node_idx, neg_he_idx,
           W_q, b_q, W_n, b_n, W_out, b_out):
    x2 = x.astype(jnp.bfloat16).reshape(2 * N, HALF)
    pn = _prep_node(pos_node_idx)
    ph = _prep_he(pos_he_idx)
    nn = _prep_node(neg_node_idx)
    nh = _prep_he(neg_he_idx)

    pos_lo, pos_hi, neg_lo, neg_hi = _sum_call(x2, pn, ph, nn, nh)
    deg_pos, deg_neg = _deg_call(ph, nh)

    return _tc_call(x, pos_lo, pos_hi, neg_lo, neg_hi, deg_pos, deg_neg,
                    W_q, b_q.reshape(1, D), W_n, b_n.reshape(1, D),
                    W_out, b_out.reshape(1, EMB))
